# baseline (speedup 1.0000x reference)
import functools

import jax
import jax.numpy as jnp
from jax import lax
from jax.experimental import pallas as pl
from jax.experimental.pallas import tpu as pltpu
from jax.experimental.pallas import tpu_sc as plsc

B = 16384
D = 32
NC = 2
NS = 16
NW = NC * NS
BPW = B // NW
CH = 128
NCH = BPW // CH

TB = 2048


def _sc_gather_body(uidx_hbm, tidx_hbm, utab_hbm, ttab_hbm,
                    ue_hbm, te_hbm,
                    uidx_v, tidx_v, urows_v, trows_v, sem):
    wid = lax.axis_index("s") * NC + lax.axis_index("c")
    base = wid * BPW
    pltpu.sync_copy(uidx_hbm.at[wid], uidx_v)
    pltpu.sync_copy(tidx_hbm.at[wid], tidx_v)
    handles = []
    for j in range(NCH):
        handles.append(pltpu.async_copy(
            utab_hbm.at[uidx_v.at[j]], urows_v.at[pl.ds(j * CH, CH)], sem))
        handles.append(pltpu.async_copy(
            ttab_hbm.at[tidx_v.at[j]], trows_v.at[pl.ds(j * CH, CH)], sem))
    for h in handles:
        h.wait()
    pltpu.sync_copy(urows_v, ue_hbm.at[pl.ds(base, BPW)])
    pltpu.sync_copy(trows_v, te_hbm.at[pl.ds(base, BPW)])


def _sc_gather(user_idx, track_idx, user_table, track_table):
    mesh = plsc.VectorSubcoreMesh(core_axis_name="c", subcore_axis_name="s")
    k = functools.partial(
        pl.kernel,
        mesh=mesh,
        out_type=(
            jax.ShapeDtypeStruct((B, D), jnp.float32),
            jax.ShapeDtypeStruct((B, D), jnp.float32),
        ),
        scratch_types=[
            pltpu.VMEM((NCH, CH), jnp.int32),
            pltpu.VMEM((NCH, CH), jnp.int32),
            pltpu.VMEM((BPW, D), jnp.float32),
            pltpu.VMEM((BPW, D), jnp.float32),
            pltpu.SemaphoreType.DMA,
        ],
        compiler_params=pltpu.CompilerParams(use_tc_tiling_on_sc=False),
    )(_sc_gather_body)
    uidx = user_idx.astype(jnp.int32).reshape(NW, NCH, CH)
    tidx = track_idx.astype(jnp.int32).reshape(NW, NCH, CH)
    return k(uidx, tidx, user_table, track_table)


def _mlp_body(ue_ref, te_ref, W1_ref, b1_ref, W2_ref, b2_ref, W3_ref, b3_ref,
              out_ref):
    x1 = ue_ref[...]
    x2 = te_ref[...]
    W1a = W1_ref[:D, :]
    W1b = W1_ref[D:, :]
    h = (jnp.dot(x1, W1a, preferred_element_type=jnp.float32)
         + jnp.dot(x2, W1b, preferred_element_type=jnp.float32)
         + b1_ref[...])
    h = jnp.maximum(h, 0.0)
    h = jnp.dot(h, W2_ref[...], preferred_element_type=jnp.float32) + b2_ref[...]
    h = jnp.maximum(h, 0.0)
    out_ref[...] = (jnp.dot(h, W3_ref[...], preferred_element_type=jnp.float32)
                    + b3_ref[...])


def _mlp(ue, te, W1, b1, W2, b2, W3, b3):
    grid = (B // TB,)
    out = pl.pallas_call(
        _mlp_body,
        grid=grid,
        in_specs=[
            pl.BlockSpec((TB, D), lambda i: (i, 0)),
            pl.BlockSpec((TB, D), lambda i: (i, 0)),
            pl.BlockSpec((2 * D, 64), lambda i: (0, 0)),
            pl.BlockSpec((1, 64), lambda i: (0, 0)),
            pl.BlockSpec((64, D), lambda i: (0, 0)),
            pl.BlockSpec((1, D), lambda i: (0, 0)),
            pl.BlockSpec((D, 1), lambda i: (0, 0)),
            pl.BlockSpec((1, 1), lambda i: (0, 0)),
        ],
        out_specs=pl.BlockSpec((TB, 1), lambda i: (i, 0)),
        out_shape=jax.ShapeDtypeStruct((B, 1), jnp.float32),
    )(ue, te, W1, b1.reshape(1, 64), W2, b2.reshape(1, D), W3,
      b3.reshape(1, 1))
    return out.reshape(B)


def kernel(user_idx, track_idx, user_table, track_table, W1, b1, W2, b2, W3, b3):
    ue, te = _sc_gather(user_idx, track_idx, user_table, track_table)
    return _mlp(ue, te, W1, b1, W2, b2, W3, b3)
